# Initial kernel scaffold; baseline (speedup 1.0000x reference)
import jax
import jax.numpy as jnp
from jax.experimental import pallas as pl


def kernel(output, target, one_hot):
    raise NotImplementedError("write your pallas kernel here")


# speedup vs baseline: 9.9918x; 9.9918x over previous
import functools

import jax
import jax.numpy as jnp
from jax.experimental import pallas as pl
from jax.experimental.pallas import tpu as pltpu

V = 32000
B = 4096
LABEL_SMOOTHING = 0.1
CONFIDENCE = 1.0 - LABEL_SMOOTHING
IGNORE_COL = (-100) % V
SMOOTH = LABEL_SMOOTHING / (V - 2)

ROWS_PER_BLOCK = 128


def _loss_block_kernel(x_ref, t_ref, out_ref):
    i = pl.program_id(0)
    x = x_ref[...]
    t = t_ref[0, 0, :]
    r = x.shape[0]

    m = jnp.max(x, axis=1, keepdims=True)
    se = jnp.sum(jnp.exp(x - m), axis=1)
    lse = m[:, 0] + jnp.log(se)
    sx = jnp.sum(x, axis=1)

    col = jax.lax.broadcasted_iota(jnp.int32, (r, V), 1)
    x_t = jnp.sum(jnp.where(col == t[:, None], x, 0.0), axis=1)
    x_i = x[:, IGNORE_COL]

    logp_t = x_t - lse
    logp_i = x_i - lse
    is_i = (t == IGNORE_COL).astype(jnp.float32)

    slog_s = SMOOTH * jnp.log(SMOOTH)
    clog_c = CONFIDENCE * jnp.log(CONFIDENCE)
    base = (V - 2 + is_i) * slog_s + clog_c
    cross = (SMOOTH * (sx - V * lse)
             + (CONFIDENCE - SMOOTH) * logp_t
             - SMOOTH * logp_i
             + is_i * SMOOTH * logp_i)
    partial = jnp.sum(base - cross)

    @pl.when(i == 0)
    def _init():
        out_ref[0, 0] = 0.0

    out_ref[0, 0] += partial


@jax.jit
def kernel(output, target, one_hot):
    del one_hot
    b, v = output.shape
    r = ROWS_PER_BLOCK
    grid = b // r
    t3 = target.astype(jnp.int32).reshape(grid, 1, r)
    total = pl.pallas_call(
        _loss_block_kernel,
        grid=(grid,),
        in_specs=[
            pl.BlockSpec((r, v), lambda i: (i, 0)),
            pl.BlockSpec((1, 1, r), lambda i: (i, 0, 0)),
        ],
        out_specs=pl.BlockSpec(memory_space=pltpu.SMEM),
        out_shape=jax.ShapeDtypeStruct((1, 1), jnp.float32),
    )(output, t3)
    return (total[0, 0] / b).astype(jnp.float32)
